# Initial kernel scaffold; baseline (speedup 1.0000x reference)
import jax
import jax.numpy as jnp
from jax.experimental import pallas as pl


def kernel(model_indices, id_table, capability_vectors, cp_w1, cp_b1, cp_w2, cp_b2, fp_w1, fp_b1, fp_w2, fp_b2):
    raise NotImplementedError("write your pallas kernel here")


# speedup vs baseline: 3.0636x; 3.0636x over previous
import functools

import jax
import jax.numpy as jnp
from jax import lax
from jax.experimental import pallas as pl
from jax.experimental.pallas import tpu as pltpu
from jax.experimental.pallas import tpu_sc as plsc

_N_MODELS = 1000
_CAP = 64
_EMB = 512
_HALF = 256
_BATCH = 16384

_info = plsc.get_sparse_core_info()
_NC = _info.num_cores
_NS = _info.num_subcores
_NW = _NC * _NS
_CH = 128
_NCHUNK = _BATCH // (_NW * _CH)


def _table_body(idt, cap, w1, b1, w2, b2, fw1, fb1, fw2, fb2, out):
    h = jnp.maximum(
        jnp.dot(cap[...], w1[...], preferred_element_type=jnp.float32) + b1[...],
        0.0,
    )
    cap_emb = jnp.dot(h, w2[...], preferred_element_type=jnp.float32) + b2[...]
    me = jnp.concatenate([idt[...], cap_emb], axis=1)
    h2 = jnp.maximum(
        jnp.dot(me, fw1[...], preferred_element_type=jnp.float32) + fb1[...],
        0.0,
    )
    out[...] = jnp.dot(h2, fw2[...], preferred_element_type=jnp.float32) + fb2[...]


def _build_table(id_table, cap_vecs, w1, b1, w2, b2, fw1, fb1, fw2, fb2):
    return pl.pallas_call(
        _table_body,
        out_shape=jax.ShapeDtypeStruct((_N_MODELS, _EMB), jnp.float32),
    )(id_table, cap_vecs, w1, b1, w2, b2, fw1, fb1, fw2, fb2)


@functools.partial(
    pl.kernel,
    out_type=jax.ShapeDtypeStruct((_BATCH, _EMB), jnp.float32),
    mesh=plsc.VectorSubcoreMesh(core_axis_name="c", subcore_axis_name="s"),
    scratch_types=[
        pltpu.VMEM((_NCHUNK, _CH), jnp.int32),
        pltpu.VMEM((_CH, _EMB), jnp.float32),
        pltpu.SemaphoreType.DMA,
    ],
)
def _sc_gather(table_hbm, idx_hbm, out_hbm, idx_v, rows_v, sem):
    wid = lax.axis_index("s") * _NC + lax.axis_index("c")
    base = wid * (_NCHUNK * _CH)
    pltpu.sync_copy(idx_hbm.at[wid], idx_v)
    for c in range(_NCHUNK):
        pltpu.async_copy(table_hbm.at[idx_v.at[c]], rows_v, sem).wait()
        pltpu.sync_copy(rows_v, out_hbm.at[pl.ds(base + c * _CH, _CH)])


def kernel(model_indices, id_table, capability_vectors, cp_w1, cp_b1, cp_w2,
           cp_b2, fp_w1, fp_b1, fp_w2, fp_b2):
    table = _build_table(
        id_table, capability_vectors,
        cp_w1, cp_b1.reshape(1, _HALF), cp_w2, cp_b2.reshape(1, _HALF),
        fp_w1, fp_b1.reshape(1, _EMB), fp_w2, fp_b2.reshape(1, _EMB),
    )
    idx3 = model_indices.astype(jnp.int32).reshape(_NW, _NCHUNK, _CH)
    return _sc_gather(table, idx3)
